# Initial kernel scaffold; baseline (speedup 1.0000x reference)
import jax
import jax.numpy as jnp
from jax.experimental import pallas as pl


def kernel(table_tiles, table_t, table_yx, table_pad, hand_tiles, hand_t, hand_yx, hand_pad, token_x, table_cursor_yx, table_cursor_p, hand_cursor_yx, hand_cursor_p, token_t, token_pad, tile_W, tile_b, token_emb, table_cursor_emb, table_pol_emb, hand_cursor_emb, hand_pol_emb, spatial_pos, temporal_pos):
    raise NotImplementedError("write your pallas kernel here")


# speedup vs baseline: 83.8612x; 83.8612x over previous
import jax
import jax.numpy as jnp
from jax.experimental import pallas as pl
from jax.experimental.pallas import tpu as pltpu

TABLE_W = 16
HAND_W = 8
C = 768
ST, SH, SK = 512, 128, 512
OUT_LEN = ST + SH + 3 * SK


def _build_kernel(tpad, hpad, kpad,
                  tt_s, tyx_s, ht_s, hyx_s, tokx_s, tokt_s, tcyx_s, hcyx_s,
                  ttv, htv, tokv, tcp_v, hcp_v,
                  ttiles_hbm, htiles_hbm,
                  w_ref, b_ref, tokemb, tcemb, tpol, hcemb, hpol, spat, temp,
                  x_hbm, t_out,
                  xbuf, tscr, hscr, ptbuf, sbuf, tsc,
                  sem_t, sem_h, sem_o):
    i = pl.program_id(0)
    tp = tpad[i]
    hp = hpad[i]
    kp = kpad[i]

    cp_t = pltpu.make_async_copy(ttiles_hbm.at[:, i, :], tscr, sem_t)
    cp_t.start()
    cp_h = pltpu.make_async_copy(htiles_hbm.at[:, i, :], hscr, sem_h)
    cp_h.start()

    xbuf[...] = jnp.zeros((OUT_LEN + 16, C), jnp.float32)

    def pt_body(j, _):
        ptbuf[pl.ds(j, 1), :] = temp[pl.ds(tokt_s[0, 0, j], 1), :]
        return _
    jax.lax.fori_loop(0, SK, pt_body, None, unroll=4)

    mask_k = jax.lax.broadcasted_iota(jnp.int32, (SK, 1), 0) < kp

    def place(data, valid, off, length):
        a = pl.multiple_of((off // 8) * 8, 8)
        s = off - a
        ext = jnp.concatenate([data, jnp.zeros((8, C), jnp.float32)], axis=0)
        ext = pltpu.roll(ext, s, 0)
        ii = jax.lax.broadcasted_iota(jnp.int32, (length + 8, 1), 0)
        m = jnp.logical_and(ii >= s, ii < s + valid)
        cur = xbuf[pl.ds(a, length + 8), :]
        xbuf[pl.ds(a, length + 8), :] = jnp.where(m, ext, cur)

    def place_t(seg, valid, off, length):
        a = pl.multiple_of((off // 128) * 128, 128)
        s = off - a
        ext = jnp.concatenate([seg, jnp.zeros((1, 128), jnp.int32)], axis=1)
        ext = pltpu.roll(ext, s, 1)
        ii = jax.lax.broadcasted_iota(jnp.int32, (1, length + 128), 1)
        m = jnp.logical_and(ii >= s, ii < s + valid)
        cur = tsc[0:1, pl.ds(a, length + 128)]
        tsc[0:1, pl.ds(a, length + 128)] = jnp.where(m, ext, cur)

    cp_t.wait()
    sbuf[...] = jnp.dot(tscr[...], w_ref[...],
                        preferred_element_type=jnp.float32) + b_ref[...]

    def s0_body(j, _):
        tv = tt_s[0, 0, j]
        yv = tyx_s[0, j, 0] * TABLE_W + tyx_s[0, j, 1]
        sbuf[pl.ds(j, 1), :] += temp[pl.ds(tv, 1), :] + spat[pl.ds(yv, 1), :]
        return _
    jax.lax.fori_loop(0, ST, s0_body, None, unroll=4)

    mask0 = jax.lax.broadcasted_iota(jnp.int32, (ST, 1), 0) < tp
    xbuf[0:ST, :] = jnp.where(mask0, sbuf[...], 0.0)

    cp_h.wait()
    sbuf[0:SH, :] = jnp.dot(hscr[...], w_ref[...],
                            preferred_element_type=jnp.float32) + b_ref[...]

    def s1_body(j, _):
        tv = ht_s[0, 0, j]
        yv = hyx_s[0, j, 0] * HAND_W + hyx_s[0, j, 1]
        sbuf[pl.ds(j, 1), :] += temp[pl.ds(tv, 1), :] + spat[pl.ds(yv, 1), :]
        return _
    jax.lax.fori_loop(0, SH, s1_body, None, unroll=4)

    place(sbuf[0:SH, :], hp, tp, SH)

    def s2_body(j, _):
        sbuf[pl.ds(j, 1), :] = tokemb[pl.ds(tokx_s[0, 0, j], 1), :]
        return _
    jax.lax.fori_loop(0, SK, s2_body, None, unroll=4)

    o2 = tp + hp
    place(sbuf[...] + ptbuf[...], kp, o2, SK)

    def s3_body(j, _):
        sbuf[pl.ds(j, 1), :] = tcemb[pl.ds(tcyx_s[0, 0, j], 1), :]
        return _
    jax.lax.fori_loop(0, SK, s3_body, None, unroll=4)

    pol_t = jnp.where(tcp_v[0] == 0, tpol[0:1, :], tpol[1:2, :])
    o3 = o2 + kp
    place(sbuf[...] + pol_t + ptbuf[...], kp, o3, SK)

    def s4_body(j, _):
        sbuf[pl.ds(j, 1), :] = hcemb[pl.ds(hcyx_s[0, 0, j], 1), :]
        return _
    jax.lax.fori_loop(0, SK, s4_body, None, unroll=4)

    pol_h = jnp.where(hcp_v[0] == 0, hpol[0:1, :], hpol[1:2, :])
    o4 = o3 + kp
    place(sbuf[...] + pol_h + ptbuf[...], kp, o4, SK)

    cp_o = pltpu.make_async_copy(xbuf.at[pl.ds(0, OUT_LEN)], x_hbm.at[:, i, :],
                                 sem_o)
    cp_o.start()

    tsc[...] = jnp.zeros_like(tsc)
    lane_t = jax.lax.broadcasted_iota(jnp.int32, (1, ST), 1)
    tsc[0:1, 0:ST] = jnp.where(lane_t < tp, ttv[0], 0)
    place_t(htv[0], hp, tp, SH)
    tok_row = tokv[0]
    place_t(tok_row, kp, o2, SK)
    place_t(tok_row, kp, o3, SK)
    place_t(tok_row, kp, o4, SK)
    t_out[0] = tsc[0:1, 0:OUT_LEN]

    cp_o.wait()


def kernel(table_tiles, table_t, table_yx, table_pad, hand_tiles, hand_t,
           hand_yx, hand_pad, token_x, table_cursor_yx, table_cursor_p,
           hand_cursor_yx, hand_cursor_p, token_t, token_pad, tile_W, tile_b,
           token_emb, table_cursor_emb, table_pol_emb, hand_cursor_emb,
           hand_pol_emb, spatial_pos, temporal_pos):
    B = table_pad.shape[0]
    i32 = jnp.int32

    tt_T = table_t.astype(i32).T
    tyx_T = jnp.transpose(table_yx.astype(i32), (1, 0, 2))
    ht_T = hand_t.astype(i32).T
    hyx_T = jnp.transpose(hand_yx.astype(i32), (1, 0, 2))
    tokx_T = token_x.astype(i32).T
    tokt_T = token_t.astype(i32).T
    tcyx_T = table_cursor_yx.astype(i32).T
    hcyx_T = hand_cursor_yx.astype(i32).T
    tcp_T = table_cursor_p.astype(i32).T[:, :, None]
    hcp_T = hand_cursor_p.astype(i32).T[:, :, None]
    ttv = tt_T[:, None, :]
    htv = ht_T[:, None, :]
    tokv = tokt_T[:, None, :]
    b2 = tile_b[None, :]
    tpad = table_pad.astype(i32)
    hpad = hand_pad.astype(i32)
    kpad = token_pad.astype(i32)

    smem = pltpu.SMEM
    vmem = pltpu.VMEM

    def smem_full():
        return pl.BlockSpec(memory_space=smem)

    def smem_blk(*shape):
        nd = len(shape)
        return pl.BlockSpec((1,) + shape,
                            lambda i: (i,) + (0,) * nd, memory_space=smem)

    def vmem_blk(*shape):
        nd = len(shape)
        return pl.BlockSpec((1,) + shape, lambda i: (i,) + (0,) * nd)

    def resident(arr):
        nd = arr.ndim
        return pl.BlockSpec(arr.shape, lambda i: (0,) * nd)

    in_specs = [
        smem_full(), smem_full(), smem_full(),
        smem_blk(1, ST), smem_blk(ST, 2), smem_blk(1, SH), smem_blk(SH, 2),
        smem_blk(1, SK), smem_blk(1, SK), smem_blk(1, SK), smem_blk(1, SK),
        vmem_blk(1, ST), vmem_blk(1, SH), vmem_blk(1, SK),
        vmem_blk(SK, 1), vmem_blk(SK, 1),
        pl.BlockSpec(memory_space=pl.ANY),
        pl.BlockSpec(memory_space=pl.ANY),
        resident(tile_W), resident(b2),
        resident(token_emb), resident(table_cursor_emb),
        resident(table_pol_emb), resident(hand_cursor_emb),
        resident(hand_pol_emb), resident(spatial_pos), resident(temporal_pos),
    ]
    out_shape = [
        jax.ShapeDtypeStruct((OUT_LEN, B, C), jnp.float32),
        jax.ShapeDtypeStruct((B, 1, OUT_LEN), i32),
    ]
    out_specs = [
        pl.BlockSpec(memory_space=pl.ANY),
        pl.BlockSpec((1, 1, OUT_LEN), lambda i: (i, 0, 0)),
    ]
    scratch_shapes = [
        pltpu.VMEM((OUT_LEN + 16, C), jnp.float32),
        pltpu.VMEM((ST, C), jnp.float32),
        pltpu.VMEM((SH, C), jnp.float32),
        pltpu.VMEM((SK, C), jnp.float32),
        pltpu.VMEM((ST, C), jnp.float32),
        pltpu.VMEM((1, OUT_LEN + 256), jnp.int32),
        pltpu.SemaphoreType.DMA,
        pltpu.SemaphoreType.DMA,
        pltpu.SemaphoreType.DMA,
    ]

    x, t3 = pl.pallas_call(
        _build_kernel,
        grid=(B,),
        in_specs=in_specs,
        out_specs=out_specs,
        out_shape=out_shape,
        scratch_shapes=scratch_shapes,
    )(tpad, hpad, kpad,
      tt_T[:, None, :], tyx_T, ht_T[:, None, :], hyx_T,
      tokx_T[:, None, :], tokt_T[:, None, :],
      tcyx_T[:, None, :], hcyx_T[:, None, :],
      ttv, htv, tokv, tcp_T, hcp_T,
      table_tiles, hand_tiles,
      tile_W, b2, token_emb, table_cursor_emb, table_pol_emb,
      hand_cursor_emb, hand_pol_emb, spatial_pos, temporal_pos)

    t = t3[:, 0, :].T
    pad = (tpad + hpad + 3 * kpad)
    return (x, t, pad)
